# baseline (speedup 1.0000x reference)
import functools
import jax
import jax.numpy as jnp
from jax import lax
from jax.experimental import pallas as pl
from jax.experimental.pallas import tpu as pltpu
from jax.experimental.pallas import tpu_sc as plsc

_N, _L, _C = 16, 4096, 64
_NC, _NS = 2, 16
_NW = _NC * _NS
_TH = _L // 2
_CT = 64
_NCH = _TH // _CT
_NBUF = 4


def _sc_partial_body(p_hbm, l_hbm, sum_out, cnt_out, pbuf, lbuf, sres, cres, *sems):
    c = lax.axis_index("c")
    s = lax.axis_index("s")
    wid = s * _NC + c
    n = wid // 2
    t_base = (wid % 2) * _TH
    psems = sems[:_NBUF]
    lsems = sems[_NBUF:]

    for b in range(_NBUF - 1):
        t0 = t_base + b * _CT
        pltpu.async_copy(p_hbm.at[n, pl.ds(t0, _CT)], pbuf.at[b], psems[b])
        pltpu.async_copy(l_hbm.at[n, pl.ds(t0, _CT)], lbuf.at[b], lsems[b])

    sa = jnp.zeros((16,), jnp.float32)
    ca = jnp.zeros((16,), jnp.float32)
    for g in range(_NCH):
        b = g % _NBUF
        gn = g + _NBUF - 1
        if gn < _NCH:
            bn = gn % _NBUF
            tn = t_base + gn * _CT
            pltpu.async_copy(p_hbm.at[n, pl.ds(tn, _CT)], pbuf.at[bn], psems[bn])
            pltpu.async_copy(l_hbm.at[n, pl.ds(tn, _CT)], lbuf.at[bn], lsems[bn])
        t0 = t_base + g * _CT
        pltpu.make_async_copy(
            p_hbm.at[n, pl.ds(t0, _CT)], pbuf.at[b], psems[b]
        ).wait()
        pltpu.make_async_copy(
            l_hbm.at[n, pl.ds(t0, _CT)], lbuf.at[b], lsems[b]
        ).wait()

        def body(i, carry, _b=b):
            s0, s1, s2, s3, c0, c1, c2, c3 = carry
            accs = [s0, s1, s2, s3]
            cnts = [c0, c1, c2, c3]
            for k in range(4):
                p = pbuf[_b, i, pl.ds(k * 16, 16)]
                l = lbuf[_b, i, pl.ds(k * 16, 16)]
                nan = l != l
                d = jnp.where(nan, 0.0, p - l)
                accs[k] = accs[k] + d * d
                cnts[k] = cnts[k] + jnp.where(nan, 0.0, 1.0)
            return (*accs, *cnts)

        z = jnp.zeros((16,), jnp.float32)
        s0, s1, s2, s3, c0, c1, c2, c3 = lax.fori_loop(
            0, _CT, body, (z, z, z, z, z, z, z, z)
        )
        sa = sa + (s0 + s1) + (s2 + s3)
        ca = ca + (c0 + c1) + (c2 + c3)

    sres[...] = sa
    cres[...] = ca
    pltpu.sync_copy(sres, sum_out.at[pl.ds(wid * 16, 16)])
    pltpu.sync_copy(cres, cnt_out.at[pl.ds(wid * 16, 16)])


@functools.cache
def _sc_partial():
    return pl.kernel(
        _sc_partial_body,
        mesh=plsc.VectorSubcoreMesh(core_axis_name="c", subcore_axis_name="s"),
        out_type=[
            jax.ShapeDtypeStruct((_NW * 16,), jnp.float32),
            jax.ShapeDtypeStruct((_NW * 16,), jnp.float32),
        ],
        scratch_types=[
            pltpu.VMEM((_NBUF, _CT, _C), jnp.float32),
            pltpu.VMEM((_NBUF, _CT, _C), jnp.float32),
            pltpu.VMEM((16,), jnp.float32),
            pltpu.VMEM((16,), jnp.float32),
        ]
        + [pltpu.SemaphoreType.DMA] * (2 * _NBUF),
        compiler_params=pltpu.CompilerParams(use_tc_tiling_on_sc=True),
    )


def _fin_body(s_ref, c_ref, o_ref):
    o_ref[0] = jnp.sum(s_ref[...]) / jnp.sum(c_ref[...])


def kernel(preds, labels):
    sums, cnts = _sc_partial()(preds, labels)
    out = pl.pallas_call(
        _fin_body,
        out_specs=pl.BlockSpec(memory_space=pltpu.SMEM),
        out_shape=jax.ShapeDtypeStruct((1,), jnp.float32),
    )(sums.reshape(_NW, 16), cnts.reshape(_NW, 16))
    return out[0]


# speedup vs baseline: 3.5141x; 3.5141x over previous
import jax
import jax.numpy as jnp
from jax.experimental import pallas as pl
from jax.experimental.pallas import tpu as pltpu

_N, _L, _C = 16, 4096, 64
_BT = 2048


def _body(p_ref, l_ref, out_ref, acc_ref):
    i = pl.program_id(0)
    j = pl.program_id(1)
    step = i * pl.num_programs(1) + j

    @pl.when(step == 0)
    def _init():
        acc_ref[0] = 0.0
        acc_ref[1] = 0.0

    l = l_ref[...]
    p = p_ref[...]
    nan = jnp.isnan(l)
    d = jnp.where(nan, 0.0, p - l)
    acc_ref[0] += jnp.sum(d * d)
    acc_ref[1] += jnp.sum(jnp.where(nan, 0.0, 1.0))

    @pl.when(step == pl.num_programs(0) * pl.num_programs(1) - 1)
    def _fin():
        out_ref[0] = acc_ref[0] / acc_ref[1]


def kernel(preds, labels):
    pt = preds.swapaxes(1, 2)
    lt = labels.swapaxes(1, 2)
    out = pl.pallas_call(
        _body,
        grid=(_N, _L // _BT),
        in_specs=[
            pl.BlockSpec((1, _C, _BT), lambda i, j: (i, 0, j)),
            pl.BlockSpec((1, _C, _BT), lambda i, j: (i, 0, j)),
        ],
        out_specs=pl.BlockSpec(memory_space=pltpu.SMEM),
        out_shape=jax.ShapeDtypeStruct((1,), jnp.float32),
        scratch_shapes=[pltpu.SMEM((2,), jnp.float32)],
    )(pt, lt)
    return out[0]


# speedup vs baseline: 5.2706x; 1.4998x over previous
import jax
import jax.numpy as jnp
from jax.experimental import pallas as pl
from jax.experimental.pallas import tpu as pltpu

_N, _L, _C = 16, 4096, 64
_BT = 2048


def _body(p_ref, l_ref, out_ref, acc_ref):
    step = pl.program_id(0)

    @pl.when(step == 0)
    def _init():
        acc_ref[0] = 0.0
        acc_ref[1] = 0.0

    l = l_ref[...]
    p = p_ref[...]
    nan = jnp.isnan(l)
    d = jnp.where(nan, 0.0, p - l)
    acc_ref[0] += jnp.sum(d * d)
    acc_ref[1] += jnp.sum(jnp.where(nan, 0.0, 1.0))

    @pl.when(step == pl.num_programs(0) - 1)
    def _fin():
        out_ref[0] = acc_ref[0] / acc_ref[1]


def kernel(preds, labels):
    pt = preds.swapaxes(1, 2)
    lt = labels.swapaxes(1, 2)
    out = pl.pallas_call(
        _body,
        grid=(_N,),
        in_specs=[
            pl.BlockSpec((1, _C, _L), lambda i: (i, 0, 0)),
            pl.BlockSpec((1, _C, _L), lambda i: (i, 0, 0)),
        ],
        out_specs=pl.BlockSpec(memory_space=pltpu.SMEM),
        out_shape=jax.ShapeDtypeStruct((1,), jnp.float32),
        scratch_shapes=[pltpu.SMEM((2,), jnp.float32)],
    )(pt, lt)
    return out[0]


# speedup vs baseline: 6.8638x; 1.3023x over previous
import jax
import jax.numpy as jnp
from jax.experimental import pallas as pl
from jax.experimental.pallas import tpu as pltpu

_N, _L, _C = 16, 4096, 64
_BN = 2


def _body(p_ref, l_ref, out_ref, acc_ref):
    step = pl.program_id(0)

    @pl.when(step == 0)
    def _init():
        acc_ref[0] = 0.0
        acc_ref[1] = 0.0

    l = l_ref[...]
    p = p_ref[...]
    nan = jnp.isnan(l)
    d = jnp.where(nan, 0.0, p - l)
    acc_ref[0] += jnp.sum(d * d)
    acc_ref[1] += jnp.sum(jnp.where(nan, 0.0, 1.0))

    @pl.when(step == pl.num_programs(0) - 1)
    def _fin():
        out_ref[0] = acc_ref[0] / acc_ref[1]


def kernel(preds, labels):
    pt = preds.swapaxes(1, 2)
    lt = labels.swapaxes(1, 2)
    out = pl.pallas_call(
        _body,
        grid=(_N // _BN,),
        in_specs=[
            pl.BlockSpec((_BN, _C, _L), lambda i: (i, 0, 0)),
            pl.BlockSpec((_BN, _C, _L), lambda i: (i, 0, 0)),
        ],
        out_specs=pl.BlockSpec(memory_space=pltpu.SMEM),
        out_shape=jax.ShapeDtypeStruct((1,), jnp.float32),
        scratch_shapes=[pltpu.SMEM((2,), jnp.float32)],
    )(pt, lt)
    return out[0]


# speedup vs baseline: 7.5374x; 1.0981x over previous
import jax
import jax.numpy as jnp
from jax.experimental import pallas as pl
from jax.experimental.pallas import tpu as pltpu

_N, _L, _C = 16, 4096, 64
_BN = 4


def _body(p_ref, l_ref, out_ref, acc_ref):
    step = pl.program_id(0)

    @pl.when(step == 0)
    def _init():
        acc_ref[0] = 0.0
        acc_ref[1] = 0.0

    l = l_ref[...]
    p = p_ref[...]
    nan = jnp.isnan(l)
    d = jnp.where(nan, 0.0, p - l)
    acc_ref[0] += jnp.sum(d * d)
    acc_ref[1] += jnp.sum(jnp.where(nan, 0.0, 1.0))

    @pl.when(step == pl.num_programs(0) - 1)
    def _fin():
        out_ref[0] = acc_ref[0] / acc_ref[1]


def kernel(preds, labels):
    pt = preds.swapaxes(1, 2)
    lt = labels.swapaxes(1, 2)
    out = pl.pallas_call(
        _body,
        grid=(_N // _BN,),
        in_specs=[
            pl.BlockSpec((_BN, _C, _L), lambda i: (i, 0, 0)),
            pl.BlockSpec((_BN, _C, _L), lambda i: (i, 0, 0)),
        ],
        out_specs=pl.BlockSpec(memory_space=pltpu.SMEM),
        out_shape=jax.ShapeDtypeStruct((1,), jnp.float32),
        scratch_shapes=[pltpu.SMEM((2,), jnp.float32)],
    )(pt, lt)
    return out[0]
